# Initial kernel scaffold; baseline (speedup 1.0000x reference)
import jax
import jax.numpy as jnp
from jax.experimental import pallas as pl


def kernel(x, edge_index, W1, b1, W2, b2):
    raise NotImplementedError("write your pallas kernel here")


# speedup vs baseline: 9.8103x; 9.8103x over previous
import dataclasses
import functools

import jax
import jax.numpy as jnp
from jax import lax
from jax.experimental import pallas as pl
from jax.experimental.pallas import tpu as pltpu
from jax.experimental.pallas import tpu_sc as plsc

NC = 2
NS = 16
LANES = 16
BATCH = 128

_mesh = functools.partial(
    plsc.VectorSubcoreMesh, core_axis_name="c", subcore_axis_name="s",
    num_cores=NC, num_subcores=NS)


def _sc_params():
  cp = pltpu.CompilerParams()
  if "needs_layout_passes" in pltpu.CompilerParams.__dataclass_fields__:
    cp = dataclasses.replace(cp, needs_layout_passes=False)
  return cp


def _ru(a, m):
  return (a + m - 1) // m * m


def _hist_call(dst2, npad):
  erows = dst2.shape[0]
  rpt = erows // (NC * NS)
  hrows = npad // 128

  @functools.partial(
      pl.kernel,
      out_type=jax.ShapeDtypeStruct((NC * NS, hrows, 128), jnp.float32),
      mesh=_mesh(),
      compiler_params=_sc_params(),
      scratch_types=[
          pltpu.VMEM((hrows, 128), jnp.float32),
          pltpu.VMEM((rpt, 128), jnp.int32),
      ],
  )
  def hist_kernel(dst_hbm, deg_hbm, hist, dbuf):
    c = lax.axis_index("c")
    s = lax.axis_index("s")
    wid = c * NS + s

    @pl.loop(0, hrows)
    def _zero(i):
      @pl.loop(0, 128 // LANES)
      def _zc(j):
        hist[i, pl.ds(j * LANES, LANES)] = jnp.zeros((LANES,), jnp.float32)

    pltpu.sync_copy(dst_hbm.at[pl.ds(wid * rpt, rpt)], dbuf)
    ones = jnp.ones((LANES,), jnp.float32)

    @pl.loop(0, rpt)
    def _row(r):
      @pl.loop(0, 128 // LANES)
      def _seg(j):
        d16 = dbuf[r, pl.ds(j * LANES, LANES)]
        plsc.addupdate_scatter(
            hist,
            [lax.shift_right_logical(d16, 7),
             lax.bitwise_and(d16, 127)],
            ones)

    pltpu.sync_copy(hist, deg_hbm.at[wid])

  return hist_kernel(dst2)


def _agg_call(srcoff, dst2, xs_flat, nchunk, npad):
  erows = dst2.shape[0]
  rpt = erows // NS
  passes = nchunk // NC
  stripe = npad // NS
  nzc = stripe // 128

  idxc = 16

  @functools.partial(
      pl.kernel,
      out_type=jax.ShapeDtypeStruct((nchunk * npad, 128), jnp.float32),
      mesh=_mesh(),
      compiler_params=_sc_params(),
      scratch_types=[
          pltpu.VMEM_SHARED((npad, 128), jnp.float32),
          pltpu.VMEM((idxc, 128), jnp.int32),
          pltpu.VMEM((idxc, 128), jnp.int32),
          pltpu.VMEM((2, BATCH, 128), jnp.float32),
          pltpu.SemaphoreType.DMA,
          pltpu.SemaphoreType.DMA,
      ],
  )
  def agg_kernel(src_hbm, dst_hbm, xs_hbm, y_hbm,
                 acc, svs, dvs, rows, semg0, semg1):
    c = lax.axis_index("c")
    s = lax.axis_index("s")

    @pl.loop(0, passes)
    def _pass(p):
      chunk = c * passes + p

      @pl.loop(0, BATCH)
      def _zr(i):
        @pl.loop(0, 128 // LANES)
        def _zc(j):
          rows[0, i, pl.ds(j * LANES, LANES)] = (
              jnp.zeros((LANES,), jnp.float32))

      @pl.loop(0, nzc)
      def _za(q):
        pltpu.sync_copy(rows.at[0], acc.at[pl.ds(s * stripe + q * 128, 128)])

      plsc.subcore_barrier()

      @pl.loop(0, rpt, step=idxc)
      def _ichunk(ib):
        pltpu.sync_copy(
            src_hbm.at[pl.ds(chunk * erows + s * rpt + ib, idxc)], svs)
        pltpu.sync_copy(dst_hbm.at[pl.ds(s * rpt + ib, idxc)], dvs)

        pltpu.async_copy(xs_hbm.at[svs.at[0]], rows.at[0], semg0)
        pltpu.async_copy(xs_hbm.at[svs.at[1]], rows.at[1], semg1)

        @pl.loop(0, idxc, step=2)
        def _b(b):
          pltpu.make_async_copy(
              xs_hbm.at[svs.at[b]], rows.at[0], semg0).wait()
          pltpu.sync_copy(rows.at[0], acc.at[dvs.at[b]], add=True)

          @pl.when(b + 2 < idxc)
          def _():
            pltpu.async_copy(xs_hbm.at[svs.at[b + 2]], rows.at[0], semg0)

          pltpu.make_async_copy(
              xs_hbm.at[svs.at[b + 1]], rows.at[1], semg1).wait()
          pltpu.sync_copy(rows.at[1], acc.at[dvs.at[b + 1]], add=True)

          @pl.when(b + 3 < idxc)
          def _():
            pltpu.async_copy(xs_hbm.at[svs.at[b + 3]], rows.at[1], semg1)

      plsc.subcore_barrier()

      pltpu.sync_copy(
          acc.at[pl.ds(s * stripe, stripe)],
          y_hbm.at[pl.ds(chunk * npad + s * stripe, stripe)])

  return agg_kernel(srcoff, dst2, xs_flat)


def _dinv_of(degs_blk):
  return lax.rsqrt(jnp.sum(degs_blk, axis=0) + 1.0)


def _scale_body(x_ref, degs_ref, o_ref):
  dinv = _dinv_of(degs_ref[...])
  o_ref[...] = (x_ref[...] * dinv)[None]


def _scale_call(x_pad, degs3, npad, f_in):
  nchunk = f_in // 128
  grid = (nchunk, npad // 128)
  return pl.pallas_call(
      _scale_body,
      grid=grid,
      in_specs=[
          pl.BlockSpec((128, 128), lambda c, i: (i, c)),
          pl.BlockSpec((NC * NS, 128, 1), lambda c, i: (0, i, 0)),
      ],
      out_specs=pl.BlockSpec((1, 128, 128), lambda c, i: (c, i, 0)),
      out_shape=jax.ShapeDtypeStruct((nchunk, npad, 128), jnp.float32),
  )(x_pad, degs3)


def _mid_body(y_ref, xs_ref, degs_ref, w1_ref, b1_ref, w2_ref, z_ref):
  dinv = _dinv_of(degs_ref[...])
  a = jnp.concatenate(
      [y_ref[k] + xs_ref[k] for k in range(y_ref.shape[0])], axis=1)
  a = a * dinv
  h = jnp.dot(a, w1_ref[...], preferred_element_type=jnp.float32)
  h = jnp.maximum(h + b1_ref[...], 0.0)
  hw = jnp.dot(h, w2_ref[...], preferred_element_type=jnp.float32)
  z = hw * dinv
  for k in range(z_ref.shape[0]):
    z_ref[k] = z[:, k * 128:(k + 1) * 128]


def _mid_call(y1, xs3, degs3, W1, b1r, W2, npad):
  c_in = y1.shape[0]
  c_out = W2.shape[1] // 128
  rb = 256
  grid = (npad // rb,)
  return pl.pallas_call(
      _mid_body,
      grid=grid,
      in_specs=[
          pl.BlockSpec((c_in, rb, 128), lambda i: (0, i, 0)),
          pl.BlockSpec((c_in, rb, 128), lambda i: (0, i, 0)),
          pl.BlockSpec((NC * NS, rb, 1), lambda i: (0, i, 0)),
          pl.BlockSpec(W1.shape, lambda i: (0, 0)),
          pl.BlockSpec(b1r.shape, lambda i: (0, 0)),
          pl.BlockSpec(W2.shape, lambda i: (0, 0)),
      ],
      out_specs=pl.BlockSpec((c_out, rb, 128), lambda i: (0, i, 0)),
      out_shape=jax.ShapeDtypeStruct((c_out, npad, 128), jnp.float32),
  )(y1, xs3, degs3, W1, b1r, W2)


def _fin_body(y2_ref, z_ref, degs_ref, b2_ref, o_ref):
  dinv = _dinv_of(degs_ref[...])
  nc = y2_ref.shape[0]
  for k in range(nc):
    o_ref[:, k * 128:(k + 1) * 128] = (
        (y2_ref[k] + z_ref[k]) * dinv + b2_ref[0:1, k * 128:(k + 1) * 128])


def _fin_call(y2, z3, degs3, b2r, npad):
  nchunk = y2.shape[0]
  f_out = nchunk * 128
  grid = (npad // 128,)
  return pl.pallas_call(
      _fin_body,
      grid=grid,
      in_specs=[
          pl.BlockSpec((nchunk, 128, 128), lambda i: (0, i, 0)),
          pl.BlockSpec((nchunk, 128, 128), lambda i: (0, i, 0)),
          pl.BlockSpec((NC * NS, 128, 1), lambda i: (0, i, 0)),
          pl.BlockSpec(b2r.shape, lambda i: (0, 0)),
      ],
      out_specs=pl.BlockSpec((128, f_out), lambda i: (i, 0)),
      out_shape=jax.ShapeDtypeStruct((npad, f_out), jnp.float32),
  )(y2, z3, degs3, b2r)


def kernel(x, edge_index, W1, b1, W2, b2):
  n, f_in = x.shape
  f_mid = W1.shape[1]
  f_out = W2.shape[1]
  e = edge_index.shape[1]

  npad = _ru(n + NC * NS, 2048)
  epad = _ru(e, 128 * 256)
  padr = npad - n
  c1 = f_in // 128
  c2 = f_out // 128

  ei = edge_index.astype(jnp.int32)
  pad_ix = n + (jnp.arange(epad - e, dtype=jnp.int32) % padr)
  src = jnp.concatenate([ei[0], pad_ix])
  dst = jnp.concatenate([ei[1], pad_ix])
  dst2 = dst.reshape(epad // 128, 128)
  src2 = src.reshape(epad // 128, 128)
  off1 = (jnp.arange(c1, dtype=jnp.int32) * npad)[:, None, None]
  off2 = (jnp.arange(c2, dtype=jnp.int32) * npad)[:, None, None]
  srcoff1 = (src2[None] + off1).reshape(c1 * (epad // 128), 128)
  srcoff2 = (src2[None] + off2).reshape(c2 * (epad // 128), 128)

  x_pad = jnp.zeros((npad, f_in), x.dtype).at[:n].set(x)
  b1r = b1.reshape(1, f_mid)
  b2r = b2.reshape(1, f_out)

  degs = _hist_call(dst2, npad)
  degs3 = degs.reshape(NC * NS, npad, 1)
  xs3 = _scale_call(x_pad, degs3, npad, f_in)
  xs_flat = xs3.reshape(c1 * npad, 128)
  y1 = _agg_call(srcoff1, dst2, xs_flat, c1, npad)
  y13 = y1.reshape(c1, npad, 128)
  z3 = _mid_call(y13, xs3, degs3, W1, b1r, W2, npad)
  z_flat = z3.reshape(c2 * npad, 128)
  y2 = _agg_call(srcoff2, dst2, z_flat, c2, npad)
  y23 = y2.reshape(c2, npad, 128)
  out = _fin_call(y23, z3, degs3, b2r, npad)
  return out[:n]
